# baseline (speedup 1.0000x reference)
import functools

import jax
import jax.numpy as jnp
from jax import lax
from jax.experimental import pallas as pl
from jax.experimental.pallas import tpu as pltpu
from jax.experimental.pallas import tpu_sc as plsc

_NUM_CORES = 2
_NUM_SUBCORES = 16
_NW = _NUM_CORES * _NUM_SUBCORES
_HPAD = 56
_NBUF = 4
_GB = 4


def kernel(indices, weight):
    b, h = indices.shape
    vocab, d = weight.shape
    batches_per_w = b // _NW
    ntile, tail = divmod(h, 8)
    idx_pad = jnp.pad(indices.astype(jnp.int32), ((0, 0), (0, _HPAD - h)))
    idx_flat = idx_pad.reshape(b * _HPAD)
    per_w_idx = batches_per_w * _HPAD

    mesh = plsc.VectorSubcoreMesh(core_axis_name="c", subcore_axis_name="s")

    @functools.partial(
        pl.kernel,
        mesh=mesh,
        out_type=jax.ShapeDtypeStruct((b, h, d), jnp.float32),
        scratch_types=[
            pltpu.VMEM((per_w_idx,), jnp.int32),
            pltpu.VMEM((_NBUF, 8, d), jnp.float32),
            pltpu.SemaphoreType.DMA,
            pltpu.SemaphoreType.DMA,
            pltpu.SemaphoreType.DMA,
            pltpu.SemaphoreType.DMA,
        ],
    )
    def gather_rows(idx_hbm, table_hbm, out_hbm, idx_v, rows_v, s0, s1, s2, s3):
        wid = lax.axis_index("s") * _NUM_CORES + lax.axis_index("c")
        base = wid * batches_per_w
        sem_g = (s0, s1, s2, s3)
        cpb = ntile + (1 if tail else 0)
        cpg = _GB * cpb
        ngroups = batches_per_w // _GB
        pltpu.sync_copy(idx_hbm.at[pl.ds(base * _HPAD, per_w_idx)], idx_v)

        def gather_desc(j, r, buf):
            return pltpu.make_async_copy(
                table_hbm.at[idx_v.at[pl.ds(j * _HPAD + r * 8, 8)]],
                rows_v.at[buf],
                sem_g[buf],
            )

        for c in range(_NBUF):
            gather_desc(c // cpb, c % cpb, c).start()

        def body(g, carry):
            for c in range(cpg):
                buf = c % _NBUF
                j_l, r = divmod(c, cpb)
                j = g * _GB + j_l
                gather_desc(j, r, buf).wait()
                if r < ntile:
                    pltpu.sync_copy(
                        rows_v.at[buf], out_hbm.at[base + j, pl.ds(r * 8, 8)]
                    )
                else:
                    pltpu.sync_copy(
                        rows_v.at[buf].at[pl.ds(0, tail)],
                        out_hbm.at[base + j, pl.ds(ntile * 8, tail)],
                    )
                jn, rn = divmod(c + _NBUF, cpb)

                @pl.when(g * _GB + jn < batches_per_w)
                def _():
                    gather_desc(g * _GB + jn, rn, buf).start()

            return carry

        lax.fori_loop(0, ngroups, body, 0)

    return gather_rows(idx_flat, weight)


# speedup vs baseline: 3.3242x; 3.3242x over previous
import functools

import jax
import jax.numpy as jnp
from jax import lax
from jax.experimental import pallas as pl
from jax.experimental.pallas import tpu as pltpu
from jax.experimental.pallas import tpu_sc as plsc

_NUM_CORES = 2
_NUM_SUBCORES = 16
_NW = _NUM_CORES * _NUM_SUBCORES
_CHUNK = 16
_NBUF = 2


def kernel(indices, weight):
    b, h = indices.shape
    vocab, d = weight.shape
    n = b * h
    per_w = n // _NW
    nchunk = per_w // _CHUNK
    idx_flat = indices.astype(jnp.int32).T.reshape(n)

    mesh = plsc.VectorSubcoreMesh(core_axis_name="c", subcore_axis_name="s")

    @functools.partial(
        pl.kernel,
        mesh=mesh,
        out_type=jax.ShapeDtypeStruct((n, d), jnp.float32),
        scratch_types=[
            pltpu.VMEM((per_w,), jnp.int32),
            pltpu.VMEM((_NBUF, _CHUNK, d), jnp.float32),
            pltpu.SemaphoreType.DMA,
            pltpu.SemaphoreType.DMA,
            pltpu.SemaphoreType.DMA,
            pltpu.SemaphoreType.DMA,
        ],
    )
    def gather_rows(idx_hbm, table_hbm, out_hbm, idx_v, rows_v, g0, g1, o0, o1):
        wid = lax.axis_index("s") * _NUM_CORES + lax.axis_index("c")
        base = wid * per_w
        sem_g = (g0, g1)
        sem_o = (o0, o1)
        pltpu.sync_copy(idx_hbm.at[pl.ds(base, per_w)], idx_v)

        def gather(i, buf):
            pltpu.async_copy(
                table_hbm.at[idx_v.at[pl.ds(i * _CHUNK, _CHUNK)]],
                rows_v.at[buf],
                sem_g[buf],
            )

        for buf in range(_NBUF):
            gather(buf, buf)

        def body(k, carry):
            for buf in range(_NBUF):
                i = k * _NBUF + buf
                pltpu.make_async_copy(
                    table_hbm.at[idx_v.at[pl.ds(i * _CHUNK, _CHUNK)]],
                    rows_v.at[buf],
                    sem_g[buf],
                ).wait()
                pltpu.async_copy(
                    rows_v.at[buf],
                    out_hbm.at[pl.ds(base + i * _CHUNK, _CHUNK)],
                    sem_o[buf],
                ).wait()

                @pl.when(i + _NBUF < nchunk)
                def _():
                    gather(i + _NBUF, buf)

            return carry

        lax.fori_loop(0, nchunk // _NBUF, body, 0)

    out = gather_rows(idx_flat, weight)
    return out.reshape(h, b, d).transpose(1, 0, 2)
